# baseline (speedup 1.0000x reference)
import functools

import jax
import jax.numpy as jnp
import numpy as np
from jax import lax
from jax.experimental import pallas as pl
from jax.experimental.pallas import tpu as pltpu
from jax.experimental.pallas import tpu_sc as plsc

B = 8
N = 4096
C = 128
G = 1024
K = 24


def _fps_body(xs_ref, ys_ref, zs_ref, idx_ref, cx_ref, cy_ref, cz_ref):
    xs = xs_ref[...]
    ys = ys_ref[...]
    zs = zs_ref[...]
    iota = lax.broadcasted_iota(jnp.int32, (B, N), 1)
    boffs = lax.broadcasted_iota(jnp.int32, (B, 1), 0) * N

    lane = lax.broadcasted_iota(jnp.int32, (B, 128), 1)

    def body(i, carry):
        far, dist, a_idx, a_cx, a_cy, a_cz = carry
        onehot = iota == far
        cx = jnp.sum(jnp.where(onehot, xs, 0.0), axis=1, keepdims=True)
        cy = jnp.sum(jnp.where(onehot, ys, 0.0), axis=1, keepdims=True)
        cz = jnp.sum(jnp.where(onehot, zs, 0.0), axis=1, keepdims=True)
        sel = lane == i
        a_idx = jnp.where(sel, far + boffs, a_idx)
        a_cx = jnp.where(sel, cx, a_cx)
        a_cy = jnp.where(sel, cy, a_cy)
        a_cz = jnp.where(sel, cz, a_cz)
        dx = xs - cx
        dy = ys - cy
        dz = zs - cz
        d = dx * dx + dy * dy + dz * dz
        dist = jnp.minimum(dist, d)
        m = jnp.max(dist, axis=1, keepdims=True)
        far = jnp.min(jnp.where(dist == m, iota, N), axis=1, keepdims=True)
        return far, dist, a_idx, a_cx, a_cy, a_cz

    far = jnp.zeros((B, 1), jnp.int32)
    dist = jnp.full((B, N), 1e10, jnp.float32)
    zi = jnp.zeros((B, 128), jnp.int32)
    zf = jnp.zeros((B, 128), jnp.float32)
    for blk in range(G // 128):
        far, dist, a_idx, a_cx, a_cy, a_cz = lax.fori_loop(
            0, 128, body, (far, dist, zi, zf, zf, zf)
        )
        sl = pl.ds(blk * 128, 128)
        idx_ref[:, sl] = a_idx
        cx_ref[:, sl] = a_cx
        cy_ref[:, sl] = a_cy
        cz_ref[:, sl] = a_cz


def _fps(xs, ys, zs):
    return pl.pallas_call(
        _fps_body,
        out_shape=(
            jax.ShapeDtypeStruct((B, G), jnp.int32),
            jax.ShapeDtypeStruct((B, G), jnp.float32),
            jax.ShapeDtypeStruct((B, G), jnp.float32),
            jax.ShapeDtypeStruct((B, G), jnp.float32),
        ),
    )(xs, ys, zs)


GT = 256


NCH = N // 128
P8 = 8


def _knn_body(c_ref, p_ref, out_ref, dist_ref, vpl_ref, gpl_ref):
    b = pl.program_id(0)
    cen = c_ref[0]
    pts = p_ref[0]
    s_src = jnp.sum(cen * cen, axis=1, keepdims=True)
    s_dst = jnp.sum(pts * pts, axis=0, keepdims=True)
    dot = jnp.dot(cen, pts, preferred_element_type=jnp.float32)
    dist_ref[...] = (s_src + s_dst) - 2.0 * dot
    lane = lax.broadcasted_iota(jnp.int32, (GT, 128), 1)

    for j in range(P8):
        d = dist_ref[...]
        bestv = d[:, 0:128]
        bestc = jnp.zeros((GT, 128), jnp.int32)
        for c in range(1, NCH):
            vc = d[:, c * 128:(c + 1) * 128]
            lt = vc < bestv
            bestv = jnp.where(lt, vc, bestv)
            bestc = jnp.where(lt, c, bestc)
        vpl_ref[j] = bestv
        gpl_ref[j] = bestc * 128 + lane
        dist_ref[...] = jnp.concatenate(
            [jnp.where(bestc == c, 1e30, d[:, c * 128:(c + 1) * 128])
             for c in range(NCH)], axis=1)

    base = b * N
    hv = vpl_ref[0]
    hg = gpl_ref[0]
    hp = jnp.zeros((GT, 128), jnp.int32)
    for j in range(K):
        m = jnp.min(hv, axis=1, keepdims=True)
        gstar = jnp.min(jnp.where(hv == m, hg, 1 << 24), axis=1, keepdims=True)
        out_ref[0, :, pl.ds(j, 1)] = gstar + base
        win = hg == gstar
        hp = jnp.where(win, hp + 1, hp)
        nv = jnp.full((GT, 128), 1e30, jnp.float32)
        ng = jnp.full((GT, 128), 1 << 24, jnp.int32)
        for k in range(1, P8):
            kk = hp == k
            nv = jnp.where(kk, vpl_ref[k], nv)
            ng = jnp.where(kk, gpl_ref[k], ng)
        hv = jnp.where(win, nv, hv)
        hg = jnp.where(win, ng, hg)


def _knn(lc_xyz, xyzT):
    return pl.pallas_call(
        _knn_body,
        grid=(B, G // GT),
        in_specs=[
            pl.BlockSpec((1, GT, 3), lambda b, t: (b, t, 0)),
            pl.BlockSpec((1, 3, N), lambda b, t: (b, 0, 0)),
        ],
        out_specs=pl.BlockSpec((1, GT, K), lambda b, t: (b, t, 0)),
        out_shape=jax.ShapeDtypeStruct((B, G, K), jnp.int32),
        scratch_shapes=[
            pltpu.VMEM((GT, N), jnp.float32),
            pltpu.VMEM((P8, GT, 128), jnp.float32),
            pltpu.VMEM((P8, GT, 128), jnp.int32),
        ],
    )(lc_xyz, xyzT)


NC = 2
NS = 16
NW = NC * NS
CH = 128
LC_CHUNKS = (B * G) // (NW * CH)
KNN_CHUNKS = (B * G * K) // (NW * CH)
SM = 16

def _sc_gather_big_body(
    xf, rgbxf, lcidx, knnidx,
    lcx_o, lcrgbx_o, knnx_o, knnrgbx_o,
    lcidx_v, knnidx_v, bx, brgbx, sem,
):
    wid = lax.axis_index("s") * NC + lax.axis_index("c")
    pltpu.sync_copy(lcidx.at[pl.ds(wid * LC_CHUNKS, LC_CHUNKS)], lcidx_v)
    pltpu.sync_copy(knnidx.at[pl.ds(wid * KNN_CHUNKS, KNN_CHUNKS)], knnidx_v)

    def do_chunk(idxrow, obase, ox, orgbx):
        h1 = pltpu.async_copy(xf.at[idxrow], bx, sem)
        h2 = pltpu.async_copy(rgbxf.at[idxrow], brgbx, sem)
        h1.wait()
        h2.wait()
        h3 = pltpu.async_copy(bx, ox.at[pl.ds(obase, CH)], sem)
        h4 = pltpu.async_copy(brgbx, orgbx.at[pl.ds(obase, CH)], sem)
        h3.wait()
        h4.wait()

    def lc_body(cc, _):
        do_chunk(lcidx_v.at[cc], (wid * LC_CHUNKS + cc) * CH, lcx_o, lcrgbx_o)
        return 0

    def knn_body(cc, _):
        do_chunk(knnidx_v.at[cc], (wid * KNN_CHUNKS + cc) * CH,
                 knnx_o, knnrgbx_o)
        return 0

    lax.fori_loop(0, LC_CHUNKS, lc_body, 0)
    lax.fori_loop(0, KNN_CHUNKS, knn_body, 0)


NV3 = (CH * 3) // 16


def _sc_gather_small_body(
    smallf, lcidx, knnidx, rowmap, colmap,
    lcrgb_o, knnxyz_o, knnrgb_o,
    lcidx_v, knnidx_v, rows_v, cols_v, bsm, bxyz, brgb, sem,
):
    wid = lax.axis_index("s") * NC + lax.axis_index("c")
    pltpu.sync_copy(lcidx.at[pl.ds(wid * LC_CHUNKS, LC_CHUNKS)], lcidx_v)
    pltpu.sync_copy(knnidx.at[pl.ds(wid * KNN_CHUNKS, KNN_CHUNKS)], knnidx_v)
    pltpu.sync_copy(rowmap, rows_v)
    pltpu.sync_copy(colmap, cols_v)

    def extract(dst, coff):
        for v in range(NV3):
            rv = rows_v[v]
            cv = cols_v[v] + coff
            dst[pl.ds(v * 16, 16)] = plsc.load_gather(bsm, [rv, cv])

    def lc_body(cc, _):
        obase = (wid * LC_CHUNKS + cc) * CH * 3
        pltpu.async_copy(smallf.at[lcidx_v.at[cc]], bsm, sem).wait()
        extract(brgb, 3)
        pltpu.sync_copy(brgb, lcrgb_o.at[pl.ds(obase, CH * 3)])
        return 0

    def knn_body(cc, _):
        obase = (wid * KNN_CHUNKS + cc) * CH * 3
        pltpu.async_copy(smallf.at[knnidx_v.at[cc]], bsm, sem).wait()
        extract(bxyz, 0)
        extract(brgb, 3)
        h1 = pltpu.async_copy(bxyz, knnxyz_o.at[pl.ds(obase, CH * 3)], sem)
        h2 = pltpu.async_copy(brgb, knnrgb_o.at[pl.ds(obase, CH * 3)], sem)
        h1.wait()
        h2.wait()
        return 0

    lax.fori_loop(0, LC_CHUNKS, lc_body, 0)
    lax.fori_loop(0, KNN_CHUNKS, knn_body, 0)


@functools.lru_cache(maxsize=1)
def _sc_gather_fns():
    mesh = plsc.VectorSubcoreMesh(
        core_axis_name="c", subcore_axis_name="s",
        num_cores=NC, num_subcores=NS,
    )
    big = pl.kernel(
        _sc_gather_big_body,
        out_type=(
            jax.ShapeDtypeStruct((B * G, C), jnp.float32),
            jax.ShapeDtypeStruct((B * G, C), jnp.float32),
            jax.ShapeDtypeStruct((B * G * K, C), jnp.float32),
            jax.ShapeDtypeStruct((B * G * K, C), jnp.float32),
        ),
        mesh=mesh,
        scratch_types=[
            pltpu.VMEM((LC_CHUNKS, CH), jnp.int32),
            pltpu.VMEM((KNN_CHUNKS, CH), jnp.int32),
            pltpu.VMEM((CH, C), jnp.float32),
            pltpu.VMEM((CH, C), jnp.float32),
            pltpu.SemaphoreType.DMA,
        ],
    )
    small = pl.kernel(
        _sc_gather_small_body,
        out_type=(
            jax.ShapeDtypeStruct((B * G * 3,), jnp.float32),
            jax.ShapeDtypeStruct((B * G * K * 3,), jnp.float32),
            jax.ShapeDtypeStruct((B * G * K * 3,), jnp.float32),
        ),
        mesh=mesh,
        compiler_params=pltpu.CompilerParams(
            use_tc_tiling_on_sc=False, needs_layout_passes=False),
        scratch_types=[
            pltpu.VMEM((LC_CHUNKS, CH), jnp.int32),
            pltpu.VMEM((KNN_CHUNKS, CH), jnp.int32),
            pltpu.VMEM((NV3, 16), jnp.int32),
            pltpu.VMEM((NV3, 16), jnp.int32),
            pltpu.VMEM((CH, SM), jnp.float32),
            pltpu.VMEM((CH * 3,), jnp.float32),
            pltpu.VMEM((CH * 3,), jnp.float32),
            pltpu.SemaphoreType.DMA,
        ],
    )
    return big, small


def kernel(xyz, x, rgb, rgbx):
    xs = xyz[:, :, 0]
    ys = xyz[:, :, 1]
    zs = xyz[:, :, 2]
    gidx, cx, cy, cz = _fps(xs, ys, zs)
    lc_xyz = jnp.stack([cx, cy, cz], axis=-1)

    xyzT = jnp.swapaxes(xyz, 1, 2)
    knn_gidx = _knn(lc_xyz, xyzT)

    xf = x.reshape(B * N, C)
    rgbxf = rgbx.reshape(B * N, C)
    smallf = jnp.concatenate(
        [xyz, rgb, jnp.zeros((B, N, SM - 6), jnp.float32)], axis=-1
    ).reshape(B * N, SM)
    e = np.arange(CH * 3)
    rowmap = jnp.asarray((e // 3).reshape(NV3, 16), dtype=jnp.int32)
    colmap = jnp.asarray((e % 3).reshape(NV3, 16), dtype=jnp.int32)

    big, small = _sc_gather_fns()
    lcidx2 = gidx.reshape((B * G) // CH, CH)
    knnidx2 = knn_gidx.reshape((B * G * K) // CH, CH)
    lcx, lcrgbx, knnx, knnrgbx = big(xf, rgbxf, lcidx2, knnidx2)
    lcrgb, knnxyz, knnrgb = small(smallf, lcidx2, knnidx2, rowmap, colmap)

    lc_x = lcx.reshape(B, G, C)
    lc_rgb = lcrgb.reshape(B, G, 3)
    lc_rgbx = lcrgbx.reshape(B, G, C)
    knn_xyz = knnxyz.reshape(B, G, K, 3)
    knn_rgb = knnrgb.reshape(B, G, K, 3)
    knn_x = knnx.reshape(B, G, K, C)
    knn_rgbx = knnrgbx.reshape(B, G, K, C)
    return (lc_xyz, lc_x, lc_rgb, lc_rgbx, knn_xyz, knn_x, knn_rgb, knn_rgbx)


# speedup vs baseline: 1.0262x; 1.0262x over previous
import functools

import jax
import jax.numpy as jnp
import numpy as np
from jax import lax
from jax.experimental import pallas as pl
from jax.experimental.pallas import tpu as pltpu
from jax.experimental.pallas import tpu_sc as plsc

B = 8
N = 4096
C = 128
G = 1024
K = 24


def _fps_body(xs_ref, ys_ref, zs_ref, idx_ref, cx_ref, cy_ref, cz_ref):
    xs = xs_ref[...]
    ys = ys_ref[...]
    zs = zs_ref[...]
    iota = lax.broadcasted_iota(jnp.int32, (B, N), 1)
    boffs = lax.broadcasted_iota(jnp.int32, (B, 1), 0) * N

    lane = lax.broadcasted_iota(jnp.int32, (B, 128), 1)

    def body(i, carry):
        far, cx, cy, cz, dist, a_idx, a_cx, a_cy, a_cz = carry
        sel = lane == i
        a_idx = jnp.where(sel, far + boffs, a_idx)
        a_cx = jnp.where(sel, cx, a_cx)
        a_cy = jnp.where(sel, cy, a_cy)
        a_cz = jnp.where(sel, cz, a_cz)
        dx = xs - cx
        dy = ys - cy
        dz = zs - cz
        d = dx * dx + dy * dy + dz * dz
        dist = jnp.minimum(dist, d)
        m = jnp.max(dist, axis=1, keepdims=True)
        eqm = dist == m
        far = jnp.min(jnp.where(eqm, iota, N), axis=1, keepdims=True)
        cnt = jnp.sum(jnp.where(eqm, 1.0, 0.0), axis=1, keepdims=True)
        cx = jnp.sum(jnp.where(eqm, xs, 0.0), axis=1, keepdims=True)
        cy = jnp.sum(jnp.where(eqm, ys, 0.0), axis=1, keepdims=True)
        cz = jnp.sum(jnp.where(eqm, zs, 0.0), axis=1, keepdims=True)

        def fix(_):
            onehot = iota == far
            fx = jnp.sum(jnp.where(onehot, xs, 0.0), axis=1, keepdims=True)
            fy = jnp.sum(jnp.where(onehot, ys, 0.0), axis=1, keepdims=True)
            fz = jnp.sum(jnp.where(onehot, zs, 0.0), axis=1, keepdims=True)
            return fx, fy, fz

        cx, cy, cz = lax.cond(
            jnp.max(cnt) > 1.5, fix, lambda _: (cx, cy, cz), 0)
        return far, cx, cy, cz, dist, a_idx, a_cx, a_cy, a_cz

    far = jnp.zeros((B, 1), jnp.int32)
    cx = xs[:, 0:1]
    cy = ys[:, 0:1]
    cz = zs[:, 0:1]
    dist = jnp.full((B, N), 1e10, jnp.float32)
    zi = jnp.zeros((B, 128), jnp.int32)
    zf = jnp.zeros((B, 128), jnp.float32)
    carry = (far, cx, cy, cz, dist, zi, zf, zf, zf)
    for blk in range(G // 128):
        carry = lax.fori_loop(0, 128, body, carry)
        far, cx, cy, cz, dist = carry[:5]
        a_idx, a_cx, a_cy, a_cz = carry[5:]
        sl = pl.ds(blk * 128, 128)
        idx_ref[:, sl] = a_idx
        cx_ref[:, sl] = a_cx
        cy_ref[:, sl] = a_cy
        cz_ref[:, sl] = a_cz
        carry = (far, cx, cy, cz, dist, zi, zf, zf, zf)


def _fps(xs, ys, zs):
    return pl.pallas_call(
        _fps_body,
        out_shape=(
            jax.ShapeDtypeStruct((B, G), jnp.int32),
            jax.ShapeDtypeStruct((B, G), jnp.float32),
            jax.ShapeDtypeStruct((B, G), jnp.float32),
            jax.ShapeDtypeStruct((B, G), jnp.float32),
        ),
    )(xs, ys, zs)


GT = 256


NCH = N // 128
P8 = 8


def _knn_body(c_ref, p_ref, out_ref, dist_ref, vpl_ref, gpl_ref):
    b = pl.program_id(0)
    cen = c_ref[0]
    pts = p_ref[0]
    s_src = jnp.sum(cen * cen, axis=1, keepdims=True)
    s_dst = jnp.sum(pts * pts, axis=0, keepdims=True)
    dot = jnp.dot(cen, pts, preferred_element_type=jnp.float32)
    dist_ref[...] = (s_src + s_dst) - 2.0 * dot
    lane = lax.broadcasted_iota(jnp.int32, (GT, 128), 1)

    for j in range(P8):
        d = dist_ref[...]
        bestv = d[:, 0:128]
        bestc = jnp.zeros((GT, 128), jnp.int32)
        for c in range(1, NCH):
            vc = d[:, c * 128:(c + 1) * 128]
            lt = vc < bestv
            bestv = jnp.where(lt, vc, bestv)
            bestc = jnp.where(lt, c, bestc)
        vpl_ref[j] = bestv
        gpl_ref[j] = bestc * 128 + lane
        dist_ref[...] = jnp.concatenate(
            [jnp.where(bestc == c, 1e30, d[:, c * 128:(c + 1) * 128])
             for c in range(NCH)], axis=1)

    base = b * N
    hv = vpl_ref[0]
    hg = gpl_ref[0]
    hp = jnp.zeros((GT, 128), jnp.int32)
    for j in range(K):
        m = jnp.min(hv, axis=1, keepdims=True)
        gstar = jnp.min(jnp.where(hv == m, hg, 1 << 24), axis=1, keepdims=True)
        out_ref[0, :, pl.ds(j, 1)] = gstar + base
        win = hg == gstar
        hp = jnp.where(win, hp + 1, hp)
        nv = jnp.full((GT, 128), 1e30, jnp.float32)
        ng = jnp.full((GT, 128), 1 << 24, jnp.int32)
        for k in range(1, P8):
            kk = hp == k
            nv = jnp.where(kk, vpl_ref[k], nv)
            ng = jnp.where(kk, gpl_ref[k], ng)
        hv = jnp.where(win, nv, hv)
        hg = jnp.where(win, ng, hg)


def _knn(lc_xyz, xyzT):
    return pl.pallas_call(
        _knn_body,
        grid=(B, G // GT),
        in_specs=[
            pl.BlockSpec((1, GT, 3), lambda b, t: (b, t, 0)),
            pl.BlockSpec((1, 3, N), lambda b, t: (b, 0, 0)),
        ],
        out_specs=pl.BlockSpec((1, GT, K), lambda b, t: (b, t, 0)),
        out_shape=jax.ShapeDtypeStruct((B, G, K), jnp.int32),
        scratch_shapes=[
            pltpu.VMEM((GT, N), jnp.float32),
            pltpu.VMEM((P8, GT, 128), jnp.float32),
            pltpu.VMEM((P8, GT, 128), jnp.int32),
        ],
    )(lc_xyz, xyzT)


NC = 2
NS = 16
NW = NC * NS
CH = 128
LC_CHUNKS = (B * G) // (NW * CH)
KNN_CHUNKS = (B * G * K) // (NW * CH)
SM = 16

def _sc_gather_big_body(
    xf, rgbxf, lcidx, knnidx,
    lcx_o, lcrgbx_o, knnx_o, knnrgbx_o,
    lcidx_v, knnidx_v, bx, brgbx, sem,
):
    wid = lax.axis_index("s") * NC + lax.axis_index("c")
    pltpu.sync_copy(lcidx.at[pl.ds(wid * LC_CHUNKS, LC_CHUNKS)], lcidx_v)
    pltpu.sync_copy(knnidx.at[pl.ds(wid * KNN_CHUNKS, KNN_CHUNKS)], knnidx_v)

    def do_chunk(idxrow, obase, ox, orgbx):
        h1 = pltpu.async_copy(xf.at[idxrow], bx, sem)
        h2 = pltpu.async_copy(rgbxf.at[idxrow], brgbx, sem)
        h1.wait()
        h2.wait()
        h3 = pltpu.async_copy(bx, ox.at[pl.ds(obase, CH)], sem)
        h4 = pltpu.async_copy(brgbx, orgbx.at[pl.ds(obase, CH)], sem)
        h3.wait()
        h4.wait()

    def lc_body(cc, _):
        do_chunk(lcidx_v.at[cc], (wid * LC_CHUNKS + cc) * CH, lcx_o, lcrgbx_o)
        return 0

    def knn_body(cc, _):
        do_chunk(knnidx_v.at[cc], (wid * KNN_CHUNKS + cc) * CH,
                 knnx_o, knnrgbx_o)
        return 0

    lax.fori_loop(0, LC_CHUNKS, lc_body, 0)
    lax.fori_loop(0, KNN_CHUNKS, knn_body, 0)


NV3 = (CH * 3) // 16


def _sc_gather_small_body(
    smallf, lcidx, knnidx, rowmap, colmap,
    lcrgb_o, knnxyz_o, knnrgb_o,
    lcidx_v, knnidx_v, rows_v, cols_v, bsm, bxyz, brgb, sem,
):
    wid = lax.axis_index("s") * NC + lax.axis_index("c")
    pltpu.sync_copy(lcidx.at[pl.ds(wid * LC_CHUNKS, LC_CHUNKS)], lcidx_v)
    pltpu.sync_copy(knnidx.at[pl.ds(wid * KNN_CHUNKS, KNN_CHUNKS)], knnidx_v)
    pltpu.sync_copy(rowmap, rows_v)
    pltpu.sync_copy(colmap, cols_v)

    def extract(dst, coff):
        for v in range(NV3):
            rv = rows_v[v]
            cv = cols_v[v] + coff
            dst[pl.ds(v * 16, 16)] = plsc.load_gather(bsm, [rv, cv])

    def lc_body(cc, _):
        obase = (wid * LC_CHUNKS + cc) * CH * 3
        pltpu.async_copy(smallf.at[lcidx_v.at[cc]], bsm, sem).wait()
        extract(brgb, 3)
        pltpu.sync_copy(brgb, lcrgb_o.at[pl.ds(obase, CH * 3)])
        return 0

    def knn_body(cc, _):
        obase = (wid * KNN_CHUNKS + cc) * CH * 3
        pltpu.async_copy(smallf.at[knnidx_v.at[cc]], bsm, sem).wait()
        extract(bxyz, 0)
        extract(brgb, 3)
        h1 = pltpu.async_copy(bxyz, knnxyz_o.at[pl.ds(obase, CH * 3)], sem)
        h2 = pltpu.async_copy(brgb, knnrgb_o.at[pl.ds(obase, CH * 3)], sem)
        h1.wait()
        h2.wait()
        return 0

    lax.fori_loop(0, LC_CHUNKS, lc_body, 0)
    lax.fori_loop(0, KNN_CHUNKS, knn_body, 0)


@functools.lru_cache(maxsize=1)
def _sc_gather_fns():
    mesh = plsc.VectorSubcoreMesh(
        core_axis_name="c", subcore_axis_name="s",
        num_cores=NC, num_subcores=NS,
    )
    big = pl.kernel(
        _sc_gather_big_body,
        out_type=(
            jax.ShapeDtypeStruct((B * G, C), jnp.float32),
            jax.ShapeDtypeStruct((B * G, C), jnp.float32),
            jax.ShapeDtypeStruct((B * G * K, C), jnp.float32),
            jax.ShapeDtypeStruct((B * G * K, C), jnp.float32),
        ),
        mesh=mesh,
        scratch_types=[
            pltpu.VMEM((LC_CHUNKS, CH), jnp.int32),
            pltpu.VMEM((KNN_CHUNKS, CH), jnp.int32),
            pltpu.VMEM((CH, C), jnp.float32),
            pltpu.VMEM((CH, C), jnp.float32),
            pltpu.SemaphoreType.DMA,
        ],
    )
    small = pl.kernel(
        _sc_gather_small_body,
        out_type=(
            jax.ShapeDtypeStruct((B * G * 3,), jnp.float32),
            jax.ShapeDtypeStruct((B * G * K * 3,), jnp.float32),
            jax.ShapeDtypeStruct((B * G * K * 3,), jnp.float32),
        ),
        mesh=mesh,
        compiler_params=pltpu.CompilerParams(
            use_tc_tiling_on_sc=False, needs_layout_passes=False),
        scratch_types=[
            pltpu.VMEM((LC_CHUNKS, CH), jnp.int32),
            pltpu.VMEM((KNN_CHUNKS, CH), jnp.int32),
            pltpu.VMEM((NV3, 16), jnp.int32),
            pltpu.VMEM((NV3, 16), jnp.int32),
            pltpu.VMEM((CH, SM), jnp.float32),
            pltpu.VMEM((CH * 3,), jnp.float32),
            pltpu.VMEM((CH * 3,), jnp.float32),
            pltpu.SemaphoreType.DMA,
        ],
    )
    return big, small


def kernel(xyz, x, rgb, rgbx):
    xs = xyz[:, :, 0]
    ys = xyz[:, :, 1]
    zs = xyz[:, :, 2]
    gidx, cx, cy, cz = _fps(xs, ys, zs)
    lc_xyz = jnp.stack([cx, cy, cz], axis=-1)

    xyzT = jnp.swapaxes(xyz, 1, 2)
    knn_gidx = _knn(lc_xyz, xyzT)

    xf = x.reshape(B * N, C)
    rgbxf = rgbx.reshape(B * N, C)
    smallf = jnp.concatenate(
        [xyz, rgb, jnp.zeros((B, N, SM - 6), jnp.float32)], axis=-1
    ).reshape(B * N, SM)
    e = np.arange(CH * 3)
    rowmap = jnp.asarray((e // 3).reshape(NV3, 16), dtype=jnp.int32)
    colmap = jnp.asarray((e % 3).reshape(NV3, 16), dtype=jnp.int32)

    big, small = _sc_gather_fns()
    lcidx2 = gidx.reshape((B * G) // CH, CH)
    knnidx2 = knn_gidx.reshape((B * G * K) // CH, CH)
    lcx, lcrgbx, knnx, knnrgbx = big(xf, rgbxf, lcidx2, knnidx2)
    lcrgb, knnxyz, knnrgb = small(smallf, lcidx2, knnidx2, rowmap, colmap)

    lc_x = lcx.reshape(B, G, C)
    lc_rgb = lcrgb.reshape(B, G, 3)
    lc_rgbx = lcrgbx.reshape(B, G, C)
    knn_xyz = knnxyz.reshape(B, G, K, 3)
    knn_rgb = knnrgb.reshape(B, G, K, 3)
    knn_x = knnx.reshape(B, G, K, C)
    knn_rgbx = knnrgbx.reshape(B, G, K, C)
    return (lc_xyz, lc_x, lc_rgb, lc_rgbx, knn_xyz, knn_x, knn_rgb, knn_rgbx)


# speedup vs baseline: 1.0329x; 1.0066x over previous
import functools

import jax
import jax.numpy as jnp
import numpy as np
from jax import lax
from jax.experimental import pallas as pl
from jax.experimental.pallas import tpu as pltpu
from jax.experimental.pallas import tpu_sc as plsc

B = 8
N = 4096
C = 128
G = 1024
K = 24


def _fps_body(xs_ref, ys_ref, zs_ref, idx_ref, cx_ref, cy_ref, cz_ref):
    xs = xs_ref[...]
    ys = ys_ref[...]
    zs = zs_ref[...]
    iota = lax.broadcasted_iota(jnp.int32, (B, N), 1)
    boffs = lax.broadcasted_iota(jnp.int32, (B, 1), 0) * N

    lane = lax.broadcasted_iota(jnp.int32, (B, 128), 1)

    def body(i, carry):
        far, cx, cy, cz, dist, a_idx, a_cx, a_cy, a_cz = carry
        sel = lane == i
        a_idx = jnp.where(sel, far + boffs, a_idx)
        a_cx = jnp.where(sel, cx, a_cx)
        a_cy = jnp.where(sel, cy, a_cy)
        a_cz = jnp.where(sel, cz, a_cz)
        dx = xs - cx
        dy = ys - cy
        dz = zs - cz
        d = dx * dx + dy * dy + dz * dz
        dist = jnp.minimum(dist, d)
        m = jnp.max(dist, axis=1, keepdims=True)
        eqm = dist == m
        far = jnp.min(jnp.where(eqm, iota, N), axis=1, keepdims=True)
        cnt = jnp.sum(jnp.where(eqm, 1.0, 0.0), axis=1, keepdims=True)
        cx = jnp.sum(jnp.where(eqm, xs, 0.0), axis=1, keepdims=True)
        cy = jnp.sum(jnp.where(eqm, ys, 0.0), axis=1, keepdims=True)
        cz = jnp.sum(jnp.where(eqm, zs, 0.0), axis=1, keepdims=True)

        def fix(_):
            onehot = iota == far
            fx = jnp.sum(jnp.where(onehot, xs, 0.0), axis=1, keepdims=True)
            fy = jnp.sum(jnp.where(onehot, ys, 0.0), axis=1, keepdims=True)
            fz = jnp.sum(jnp.where(onehot, zs, 0.0), axis=1, keepdims=True)
            return fx, fy, fz

        cx, cy, cz = lax.cond(
            jnp.max(cnt) > 1.5, fix, lambda _: (cx, cy, cz), 0)
        return far, cx, cy, cz, dist, a_idx, a_cx, a_cy, a_cz

    far = jnp.zeros((B, 1), jnp.int32)
    cx = xs[:, 0:1]
    cy = ys[:, 0:1]
    cz = zs[:, 0:1]
    dist = jnp.full((B, N), 1e10, jnp.float32)
    zi = jnp.zeros((B, 128), jnp.int32)
    zf = jnp.zeros((B, 128), jnp.float32)
    carry = (far, cx, cy, cz, dist, zi, zf, zf, zf)
    for blk in range(G // 128):
        carry = lax.fori_loop(0, 128, body, carry)
        far, cx, cy, cz, dist = carry[:5]
        a_idx, a_cx, a_cy, a_cz = carry[5:]
        sl = pl.ds(blk * 128, 128)
        idx_ref[:, sl] = a_idx
        cx_ref[:, sl] = a_cx
        cy_ref[:, sl] = a_cy
        cz_ref[:, sl] = a_cz
        carry = (far, cx, cy, cz, dist, zi, zf, zf, zf)


def _fps(xs, ys, zs):
    return pl.pallas_call(
        _fps_body,
        out_shape=(
            jax.ShapeDtypeStruct((B, G), jnp.int32),
            jax.ShapeDtypeStruct((B, G), jnp.float32),
            jax.ShapeDtypeStruct((B, G), jnp.float32),
            jax.ShapeDtypeStruct((B, G), jnp.float32),
        ),
    )(xs, ys, zs)


GT = 256


NCH = N // 128
P8 = 8


def _knn_body(c_ref, p_ref, out_ref, dist_ref, vpl_ref, gpl_ref):
    b = pl.program_id(0)
    cen = c_ref[0]
    pts = p_ref[0]
    s_src = jnp.sum(cen * cen, axis=1, keepdims=True)
    s_dst = jnp.sum(pts * pts, axis=0, keepdims=True)
    dot = jnp.dot(cen, pts, preferred_element_type=jnp.float32)
    dist_ref[...] = (s_src + s_dst) - 2.0 * dot
    lane = lax.broadcasted_iota(jnp.int32, (GT, 128), 1)

    for j in range(P8):
        d = dist_ref[...]
        bestv = d[:, 0:128]
        bestc = jnp.zeros((GT, 128), jnp.int32)
        for c in range(1, NCH):
            vc = d[:, c * 128:(c + 1) * 128]
            lt = vc < bestv
            bestv = jnp.where(lt, vc, bestv)
            bestc = jnp.where(lt, c, bestc)
        vpl_ref[j] = bestv
        gpl_ref[j] = bestc * 128 + lane
        dist_ref[...] = jnp.concatenate(
            [jnp.where(bestc == c, 1e30, d[:, c * 128:(c + 1) * 128])
             for c in range(NCH)], axis=1)

    base = b * N
    hv = vpl_ref[0]
    hg = gpl_ref[0]
    hp = jnp.zeros((GT, 128), jnp.int32)
    for j in range(K):
        m = jnp.min(hv, axis=1, keepdims=True)
        gstar = jnp.min(jnp.where(hv == m, hg, 1 << 24), axis=1, keepdims=True)
        out_ref[0, :, pl.ds(j, 1)] = gstar + base
        win = hg == gstar
        hp = jnp.where(win, hp + 1, hp)
        nv = jnp.full((GT, 128), 1e30, jnp.float32)
        ng = jnp.full((GT, 128), 1 << 24, jnp.int32)
        for k in range(1, P8):
            kk = hp == k
            nv = jnp.where(kk, vpl_ref[k], nv)
            ng = jnp.where(kk, gpl_ref[k], ng)
        hv = jnp.where(win, nv, hv)
        hg = jnp.where(win, ng, hg)


def _knn(lc_xyz, xyzT):
    return pl.pallas_call(
        _knn_body,
        grid=(B, G // GT),
        in_specs=[
            pl.BlockSpec((1, GT, 3), lambda b, t: (b, t, 0)),
            pl.BlockSpec((1, 3, N), lambda b, t: (b, 0, 0)),
        ],
        out_specs=pl.BlockSpec((1, GT, K), lambda b, t: (b, t, 0)),
        out_shape=jax.ShapeDtypeStruct((B, G, K), jnp.int32),
        scratch_shapes=[
            pltpu.VMEM((GT, N), jnp.float32),
            pltpu.VMEM((P8, GT, 128), jnp.float32),
            pltpu.VMEM((P8, GT, 128), jnp.int32),
        ],
    )(lc_xyz, xyzT)


NC = 2
NS = 16
NW = NC * NS
CH = 128
LC_CHUNKS = (B * G) // (NW * CH)
KNN_CHUNKS = (B * G * K) // (NW * CH)
SM = 16

GCH = 4 * K
KNN_GCHUNKS = (G // NW) * B // 4


def _sc_gather_big_body(
    xf, rgbxf, lcidx, knnidx,
    lcx_o, lcrgbx_o, knnx_o, knnrgbx_o,
    lcidx_v, knnidx_v, bx, brgbx, sem,
):
    wid = lax.axis_index("s") * NC + lax.axis_index("c")
    b = wid // 4
    gbase = (wid % 4) * (G // 4)
    pltpu.sync_copy(lcidx.at[pl.ds(wid * LC_CHUNKS, LC_CHUNKS)], lcidx_v)
    pltpu.sync_copy(knnidx.at[pl.ds(wid * KNN_GCHUNKS, KNN_GCHUNKS)],
                    knnidx_v)

    def lc_body(cc, _):
        goff = gbase + cc * CH
        h1 = pltpu.async_copy(xf.at[lcidx_v.at[cc]], bx.at[pl.ds(0, CH)], sem)
        h2 = pltpu.async_copy(rgbxf.at[lcidx_v.at[cc]],
                              brgbx.at[pl.ds(0, CH)], sem)
        h1.wait()
        h2.wait()
        h3 = pltpu.async_copy(bx.at[pl.ds(0, CH)],
                              lcx_o.at[b, pl.ds(goff, CH)], sem)
        h4 = pltpu.async_copy(brgbx.at[pl.ds(0, CH)],
                              lcrgbx_o.at[b, pl.ds(goff, CH)], sem)
        h3.wait()
        h4.wait()
        return 0

    def knn_body(cc, _):
        g0 = gbase + cc * 4
        h1 = pltpu.async_copy(xf.at[knnidx_v.at[cc]], bx.at[pl.ds(0, GCH)],
                              sem)
        h2 = pltpu.async_copy(rgbxf.at[knnidx_v.at[cc]],
                              brgbx.at[pl.ds(0, GCH)], sem)
        h1.wait()
        h2.wait()
        hs = []
        for i in range(4):
            sl = pl.ds(i * K, K)
            hs.append(pltpu.async_copy(bx.at[sl], knnx_o.at[b, g0 + i], sem))
            hs.append(pltpu.async_copy(brgbx.at[sl],
                                       knnrgbx_o.at[b, g0 + i], sem))
        for h in hs:
            h.wait()
        return 0

    lax.fori_loop(0, LC_CHUNKS, lc_body, 0)
    lax.fori_loop(0, KNN_GCHUNKS, knn_body, 0)


NV3 = (CH * 3) // 16


def _sc_gather_small_body(
    smallf, lcidx, knnidx, rowmap, colmap,
    lcrgb_o, knnxyz_o, knnrgb_o,
    lcidx_v, knnidx_v, rows_v, cols_v, bsm, bxyz, brgb, sem,
):
    wid = lax.axis_index("s") * NC + lax.axis_index("c")
    pltpu.sync_copy(lcidx.at[pl.ds(wid * LC_CHUNKS, LC_CHUNKS)], lcidx_v)
    pltpu.sync_copy(knnidx.at[pl.ds(wid * KNN_CHUNKS, KNN_CHUNKS)], knnidx_v)
    pltpu.sync_copy(rowmap, rows_v)
    pltpu.sync_copy(colmap, cols_v)

    def extract(dst, coff):
        for v in range(NV3):
            rv = rows_v[v]
            cv = cols_v[v] + coff
            dst[pl.ds(v * 16, 16)] = plsc.load_gather(bsm, [rv, cv])

    def lc_body(cc, _):
        obase = (wid * LC_CHUNKS + cc) * CH * 3
        pltpu.async_copy(smallf.at[lcidx_v.at[cc]], bsm, sem).wait()
        extract(brgb, 3)
        pltpu.sync_copy(brgb, lcrgb_o.at[pl.ds(obase, CH * 3)])
        return 0

    def knn_body(cc, _):
        obase = (wid * KNN_CHUNKS + cc) * CH * 3
        pltpu.async_copy(smallf.at[knnidx_v.at[cc]], bsm, sem).wait()
        extract(bxyz, 0)
        extract(brgb, 3)
        h1 = pltpu.async_copy(bxyz, knnxyz_o.at[pl.ds(obase, CH * 3)], sem)
        h2 = pltpu.async_copy(brgb, knnrgb_o.at[pl.ds(obase, CH * 3)], sem)
        h1.wait()
        h2.wait()
        return 0

    lax.fori_loop(0, LC_CHUNKS, lc_body, 0)
    lax.fori_loop(0, KNN_CHUNKS, knn_body, 0)


@functools.lru_cache(maxsize=1)
def _sc_gather_fns():
    mesh = plsc.VectorSubcoreMesh(
        core_axis_name="c", subcore_axis_name="s",
        num_cores=NC, num_subcores=NS,
    )
    big = pl.kernel(
        _sc_gather_big_body,
        out_type=(
            jax.ShapeDtypeStruct((B, G, C), jnp.float32),
            jax.ShapeDtypeStruct((B, G, C), jnp.float32),
            jax.ShapeDtypeStruct((B, G, K, C), jnp.float32),
            jax.ShapeDtypeStruct((B, G, K, C), jnp.float32),
        ),
        mesh=mesh,
        scratch_types=[
            pltpu.VMEM((LC_CHUNKS, CH), jnp.int32),
            pltpu.VMEM((KNN_GCHUNKS, GCH), jnp.int32),
            pltpu.VMEM((CH, C), jnp.float32),
            pltpu.VMEM((CH, C), jnp.float32),
            pltpu.SemaphoreType.DMA,
        ],
    )
    small = pl.kernel(
        _sc_gather_small_body,
        out_type=(
            jax.ShapeDtypeStruct((B * G * 3,), jnp.float32),
            jax.ShapeDtypeStruct((B * G * K * 3,), jnp.float32),
            jax.ShapeDtypeStruct((B * G * K * 3,), jnp.float32),
        ),
        mesh=mesh,
        compiler_params=pltpu.CompilerParams(
            use_tc_tiling_on_sc=False, needs_layout_passes=False),
        scratch_types=[
            pltpu.VMEM((LC_CHUNKS, CH), jnp.int32),
            pltpu.VMEM((KNN_CHUNKS, CH), jnp.int32),
            pltpu.VMEM((NV3, 16), jnp.int32),
            pltpu.VMEM((NV3, 16), jnp.int32),
            pltpu.VMEM((CH, SM), jnp.float32),
            pltpu.VMEM((CH * 3,), jnp.float32),
            pltpu.VMEM((CH * 3,), jnp.float32),
            pltpu.SemaphoreType.DMA,
        ],
    )
    return big, small


def kernel(xyz, x, rgb, rgbx):
    xs = xyz[:, :, 0]
    ys = xyz[:, :, 1]
    zs = xyz[:, :, 2]
    gidx, cx, cy, cz = _fps(xs, ys, zs)
    lc_xyz = jnp.stack([cx, cy, cz], axis=-1)

    xyzT = jnp.swapaxes(xyz, 1, 2)
    knn_gidx = _knn(lc_xyz, xyzT)

    xf = x.reshape(B * N, C)
    rgbxf = rgbx.reshape(B * N, C)
    smallf = jnp.concatenate(
        [xyz, rgb, jnp.zeros((B, N, SM - 6), jnp.float32)], axis=-1
    ).reshape(B * N, SM)
    e = np.arange(CH * 3)
    rowmap = jnp.asarray((e // 3).reshape(NV3, 16), dtype=jnp.int32)
    colmap = jnp.asarray((e % 3).reshape(NV3, 16), dtype=jnp.int32)

    big, small = _sc_gather_fns()
    lcidx2 = gidx.reshape((B * G) // CH, CH)
    knnidx2 = knn_gidx.reshape((B * G * K) // CH, CH)
    knnidx3 = knn_gidx.reshape((B * G * K) // GCH, GCH)
    lc_x, lc_rgbx, knn_x, knn_rgbx = big(xf, rgbxf, lcidx2, knnidx3)
    lcrgb, knnxyz, knnrgb = small(smallf, lcidx2, knnidx2, rowmap, colmap)

    lc_rgb = lcrgb.reshape(B, G, 3)
    knn_xyz = knnxyz.reshape(B, G, K, 3)
    knn_rgb = knnrgb.reshape(B, G, K, 3)
    return (lc_xyz, lc_x, lc_rgb, lc_rgbx, knn_xyz, knn_x, knn_rgb, knn_rgbx)
